# initial kernel scaffold (unmeasured)
import jax
import jax.numpy as jnp
from jax import lax
from jax.experimental import pallas as pl
from jax.experimental.pallas import tpu as pltpu

N_DEV = 8


def kernel(x, w_mat):
    m_per, k = x.shape
    _, n = w_mat.shape
    n_per = n // N_DEV

    def body(x_ref, w_hbm, out_ref, wbuf, y_ref, q_ref, qr_ref, amax_ref,
             w_sems, own_sem, amax_send, amax_recv, ch_send, ch_recv):
        my = lax.axis_index("i")

        def w_copy(j, slot):
            return pltpu.make_async_copy(
                w_hbm.at[:, pl.ds(j * n_per, n_per)], wbuf.at[slot],
                w_sems.at[slot])

        w_copy(0, 0).start()
        local_amax = jnp.float32(0.0)
        for j in range(N_DEV):
            slot = j % 2
            if j + 1 < N_DEV:
                w_copy(j + 1, (j + 1) % 2).start()
            w_copy(j, slot).wait()
            yj = jnp.dot(x_ref[:, :], wbuf[slot],
                         preferred_element_type=jnp.float32)
            y_ref[:, pl.ds(j * n_per, n_per)] = yj
            local_amax = jnp.maximum(local_amax, jnp.max(jnp.abs(yj)))

        amax_ref[my] = jnp.full((8, 128), local_amax, jnp.float32)
        for d in range(1, N_DEV):
            t = lax.rem(my + d, N_DEV)
            pltpu.make_async_remote_copy(
                src_ref=amax_ref.at[my], dst_ref=amax_ref.at[my],
                send_sem=amax_send.at[d], recv_sem=amax_recv.at[my],
                device_id=(t,), device_id_type=pl.DeviceIdType.MESH,
            ).start()
        for d in range(1, N_DEV):
            s = lax.rem(my + d, N_DEV)
            pltpu.make_async_remote_copy(
                src_ref=amax_ref.at[my], dst_ref=amax_ref.at[s],
                send_sem=amax_send.at[d], recv_sem=amax_recv.at[s],
                device_id=(s,), device_id_type=pl.DeviceIdType.MESH,
            ).wait_recv()
        for d in range(1, N_DEV):
            pltpu.make_async_remote_copy(
                src_ref=amax_ref.at[my], dst_ref=amax_ref.at[my],
                send_sem=amax_send.at[d], recv_sem=amax_recv.at[my],
                device_id=(my,), device_id_type=pl.DeviceIdType.MESH,
            ).wait_send()

        gmax = jnp.max(amax_ref[:, :, :])
        scale = gmax / 127.0

        q_ref[:, :] = jnp.clip(jnp.round(y_ref[:, :] / scale),
                               -127.0, 127.0).astype(jnp.int8)

        own = pltpu.make_async_copy(
            q_ref.at[:, pl.ds(my * n_per, n_per)], qr_ref.at[my], own_sem)
        own.start()

        for d in range(1, N_DEV):
            t = lax.rem(my + d, N_DEV)
            pltpu.make_async_remote_copy(
                src_ref=q_ref.at[:, pl.ds(t * n_per, n_per)],
                dst_ref=qr_ref.at[my],
                send_sem=ch_send.at[d], recv_sem=ch_recv.at[my],
                device_id=(t,), device_id_type=pl.DeviceIdType.MESH,
            ).start()
        own.wait()
        for d in range(1, N_DEV):
            s = lax.rem(my + d, N_DEV)
            pltpu.make_async_remote_copy(
                src_ref=q_ref.at[:, pl.ds(0, n_per)],
                dst_ref=qr_ref.at[s],
                send_sem=ch_send.at[d], recv_sem=ch_recv.at[s],
                device_id=(s,), device_id_type=pl.DeviceIdType.MESH,
            ).wait_recv()
        for d in range(1, N_DEV):
            pltpu.make_async_remote_copy(
                src_ref=q_ref.at[:, pl.ds(0, n_per)],
                dst_ref=qr_ref.at[my],
                send_sem=ch_send.at[d], recv_sem=ch_recv.at[my],
                device_id=(my,), device_id_type=pl.DeviceIdType.MESH,
            ).wait_send()

        out_ref[:, :] = (qr_ref[:, :, :].astype(jnp.float32) * scale
                         ).reshape(N_DEV * m_per, n_per)

    return pl.pallas_call(
        body,
        out_shape=jax.ShapeDtypeStruct((N_DEV * m_per, n_per), jnp.float32),
        in_specs=[
            pl.BlockSpec(memory_space=pltpu.VMEM),
            pl.BlockSpec(memory_space=pltpu.ANY),
        ],
        out_specs=pl.BlockSpec(memory_space=pltpu.VMEM),
        scratch_shapes=[
            pltpu.VMEM((2, k, n_per), jnp.bfloat16),
            pltpu.VMEM((m_per, n), jnp.float32),
            pltpu.VMEM((m_per, n), jnp.int8),
            pltpu.VMEM((N_DEV, m_per, n_per), jnp.int8),
            pltpu.VMEM((N_DEV, 8, 128), jnp.float32),
            pltpu.SemaphoreType.DMA((2,)),
            pltpu.SemaphoreType.DMA,
            pltpu.SemaphoreType.DMA((N_DEV,)),
            pltpu.SemaphoreType.DMA((N_DEV,)),
            pltpu.SemaphoreType.DMA((N_DEV,)),
            pltpu.SemaphoreType.DMA((N_DEV,)),
        ],
        compiler_params=pltpu.CompilerParams(collective_id=0),
    )(x, w_mat)


# baseline (device time: 148562 ns/iter reference)
import jax
import jax.numpy as jnp
from jax import lax
from jax.experimental import pallas as pl
from jax.experimental.pallas import tpu as pltpu

N_DEV = 8
W_CHUNK = 256
X_CHUNK = 1024


def kernel(x, w_mat):
    m_per, k = x.shape
    _, n = w_mat.shape
    n_per = n // N_DEV
    n_wc = n // W_CHUNK
    n_xc = k // X_CHUNK

    def body(x_hbm, w_hbm, out_ref, xs_ref, xb_ref, wbuf, y_ref, q_ref,
             qr_ref, amax_ref, x_sem, w_sems, own_sem, amax_send,
             amax_recv, ch_send, ch_recv):
        my = lax.axis_index("i")

        def w_copy(j, slot):
            return pltpu.make_async_copy(
                w_hbm.at[:, pl.ds(j * W_CHUNK, W_CHUNK)], wbuf.at[slot],
                w_sems.at[slot])

        w_copy(0, 0).start()
        for c in range(n_xc):
            cp = pltpu.make_async_copy(
                x_hbm.at[:, pl.ds(c * X_CHUNK, X_CHUNK)], xs_ref, x_sem)
            cp.start()
            cp.wait()
            xb_ref[:, pl.ds(c * X_CHUNK, X_CHUNK)] = (
                xs_ref[:, :].astype(jnp.bfloat16))

        local_amax = jnp.float32(0.0)
        for j in range(n_wc):
            slot = j % 2
            if j + 1 < n_wc:
                w_copy(j + 1, (j + 1) % 2).start()
            w_copy(j, slot).wait()
            yj = jnp.dot(xb_ref[:, :], wbuf[slot].astype(jnp.bfloat16),
                         preferred_element_type=jnp.float32)
            y_ref[:, pl.ds(j * W_CHUNK, W_CHUNK)] = yj
            local_amax = jnp.maximum(local_amax, jnp.max(jnp.abs(yj)))

        amax_ref[my] = jnp.full((8, 128), local_amax, jnp.float32)
        for d in range(1, N_DEV):
            t = lax.rem(my + d, N_DEV)
            pltpu.make_async_remote_copy(
                src_ref=amax_ref.at[my], dst_ref=amax_ref.at[my],
                send_sem=amax_send.at[d], recv_sem=amax_recv.at[my],
                device_id=(t,), device_id_type=pl.DeviceIdType.MESH,
            ).start()
        for d in range(1, N_DEV):
            s = lax.rem(my + d, N_DEV)
            pltpu.make_async_remote_copy(
                src_ref=amax_ref.at[my], dst_ref=amax_ref.at[s],
                send_sem=amax_send.at[d], recv_sem=amax_recv.at[s],
                device_id=(s,), device_id_type=pl.DeviceIdType.MESH,
            ).wait_recv()
        for d in range(1, N_DEV):
            pltpu.make_async_remote_copy(
                src_ref=amax_ref.at[my], dst_ref=amax_ref.at[my],
                send_sem=amax_send.at[d], recv_sem=amax_recv.at[my],
                device_id=(my,), device_id_type=pl.DeviceIdType.MESH,
            ).wait_send()

        gmax = jnp.max(amax_ref[:, :, :])
        scale = gmax / 127.0

        for j in range(N_DEV):
            q_ref[:, pl.ds(j * n_per, n_per)] = jnp.clip(
                jnp.round(y_ref[:, pl.ds(j * n_per, n_per)] / scale),
                -127.0, 127.0).astype(jnp.int8)

        own = pltpu.make_async_copy(
            q_ref.at[:, pl.ds(my * n_per, n_per)], qr_ref.at[my], own_sem)
        own.start()

        for d in range(1, N_DEV):
            t = lax.rem(my + d, N_DEV)
            pltpu.make_async_remote_copy(
                src_ref=q_ref.at[:, pl.ds(t * n_per, n_per)],
                dst_ref=qr_ref.at[my],
                send_sem=ch_send.at[d], recv_sem=ch_recv.at[my],
                device_id=(t,), device_id_type=pl.DeviceIdType.MESH,
            ).start()
        own.wait()
        for d in range(1, N_DEV):
            s = lax.rem(my + d, N_DEV)
            pltpu.make_async_remote_copy(
                src_ref=q_ref.at[:, pl.ds(0, n_per)],
                dst_ref=qr_ref.at[s],
                send_sem=ch_send.at[d], recv_sem=ch_recv.at[s],
                device_id=(s,), device_id_type=pl.DeviceIdType.MESH,
            ).wait_recv()
        for d in range(1, N_DEV):
            pltpu.make_async_remote_copy(
                src_ref=q_ref.at[:, pl.ds(0, n_per)],
                dst_ref=qr_ref.at[my],
                send_sem=ch_send.at[d], recv_sem=ch_recv.at[my],
                device_id=(my,), device_id_type=pl.DeviceIdType.MESH,
            ).wait_send()

        for s in range(N_DEV):
            out_ref[pl.ds(s * m_per, m_per), :] = (
                qr_ref[s].astype(jnp.float32) * scale
            ).astype(jnp.bfloat16)

    return pl.pallas_call(
        body,
        out_shape=jax.ShapeDtypeStruct((N_DEV * m_per, n_per), jnp.bfloat16),
        in_specs=[
            pl.BlockSpec(memory_space=pl.ANY),
            pl.BlockSpec(memory_space=pl.ANY),
        ],
        out_specs=pl.BlockSpec(memory_space=pltpu.VMEM),
        scratch_shapes=[
            pltpu.VMEM((m_per, X_CHUNK), jnp.float32),
            pltpu.VMEM((m_per, k), jnp.bfloat16),
            pltpu.VMEM((2, k, W_CHUNK), jnp.float32),
            pltpu.VMEM((m_per, n), jnp.float32),
            pltpu.VMEM((m_per, n), jnp.int8),
            pltpu.VMEM((N_DEV, m_per, n_per), jnp.int8),
            pltpu.VMEM((N_DEV, 8, 128), jnp.float32),
            pltpu.SemaphoreType.DMA,
            pltpu.SemaphoreType.DMA((2,)),
            pltpu.SemaphoreType.DMA,
            pltpu.SemaphoreType.DMA((N_DEV,)),
            pltpu.SemaphoreType.DMA((N_DEV,)),
            pltpu.SemaphoreType.DMA((N_DEV,)),
            pltpu.SemaphoreType.DMA((N_DEV,)),
        ],
        compiler_params=pltpu.CompilerParams(
            vmem_limit_bytes=52 * 1024 * 1024),
    )(x, w_mat)


# device time: 114480 ns/iter; 1.2977x vs baseline; 1.2977x over previous
import jax
import jax.numpy as jnp
from jax import lax
from jax.experimental import pallas as pl
from jax.experimental.pallas import tpu as pltpu

N_DEV = 8
W_CHUNK = 512
X_CHUNK = 1024


def kernel(x, w_mat):
    m_per, k = x.shape
    _, n = w_mat.shape
    n_per = n // N_DEV
    n_wc = n // W_CHUNK
    n_xc = k // X_CHUNK
    per_chunk = n_per // W_CHUNK

    def body(x_hbm, w_hbm, out_ref, xs_ref, xb_ref, wbuf, y_ref, q_ref,
             qr_ref, amax_ref, x_sem, w_sems, own_sem, amax_send,
             amax_recv, ch_send, ch_recv):
        my = lax.axis_index("i")

        def w_copy(j, slot):
            return pltpu.make_async_copy(
                w_hbm.at[:, pl.ds(j * W_CHUNK, W_CHUNK)], wbuf.at[slot],
                w_sems.at[slot])

        w_copy(0, 0).start()
        for c in range(n_xc):
            cp = pltpu.make_async_copy(
                x_hbm.at[:, pl.ds(c * X_CHUNK, X_CHUNK)], xs_ref, x_sem)
            cp.start()
            cp.wait()
            xb_ref[:, pl.ds(c * X_CHUNK, X_CHUNK)] = (
                xs_ref[:, :].astype(jnp.bfloat16))

        local_amax = jnp.float32(0.0)
        for j in range(n_wc):
            slot = j % 2
            if j + 1 < n_wc:
                w_copy(j + 1, (j + 1) % 2).start()
            w_copy(j, slot).wait()
            yj = jnp.dot(xb_ref[:, :], wbuf[slot].astype(jnp.bfloat16),
                         preferred_element_type=jnp.float32)
            y_ref[j // per_chunk, :,
                  pl.ds((j % per_chunk) * W_CHUNK, W_CHUNK)] = yj
            local_amax = jnp.maximum(local_amax, jnp.max(jnp.abs(yj)))

        amax_ref[my] = jnp.full((8, 128), local_amax, jnp.float32)
        for d in range(1, N_DEV):
            t = lax.rem(my + d, N_DEV)
            pltpu.make_async_remote_copy(
                src_ref=amax_ref.at[my], dst_ref=amax_ref.at[my],
                send_sem=amax_send.at[d], recv_sem=amax_recv.at[my],
                device_id=(t,), device_id_type=pl.DeviceIdType.MESH,
            ).start()
        for d in range(1, N_DEV):
            s = lax.rem(my + d, N_DEV)
            pltpu.make_async_remote_copy(
                src_ref=amax_ref.at[my], dst_ref=amax_ref.at[s],
                send_sem=amax_send.at[d], recv_sem=amax_recv.at[s],
                device_id=(s,), device_id_type=pl.DeviceIdType.MESH,
            ).wait_recv()
        for d in range(1, N_DEV):
            pltpu.make_async_remote_copy(
                src_ref=amax_ref.at[my], dst_ref=amax_ref.at[my],
                send_sem=amax_send.at[d], recv_sem=amax_recv.at[my],
                device_id=(my,), device_id_type=pl.DeviceIdType.MESH,
            ).wait_send()

        gmax = jnp.max(amax_ref[:, :, :])
        scale = gmax / 127.0
        inv_scale = 127.0 / gmax

        def quant(t):
            q_ref[t] = jnp.clip(jnp.round(y_ref[t] * inv_scale),
                                -127.0, 127.0).astype(jnp.int8)

        for d in range(1, N_DEV):
            t = lax.rem(my + d, N_DEV)
            quant(t)
            pltpu.make_async_remote_copy(
                src_ref=q_ref.at[t], dst_ref=qr_ref.at[my],
                send_sem=ch_send.at[d], recv_sem=ch_recv.at[my],
                device_id=(t,), device_id_type=pl.DeviceIdType.MESH,
            ).start()
        quant(my)
        own = pltpu.make_async_copy(q_ref.at[my], qr_ref.at[my], own_sem)
        own.start()
        own.wait()
        out_ref[pl.ds(my * m_per, m_per), :] = (
            qr_ref[my].astype(jnp.float32) * scale).astype(jnp.bfloat16)

        for d in range(1, N_DEV):
            s = lax.rem(my + d, N_DEV)
            pltpu.make_async_remote_copy(
                src_ref=q_ref.at[0], dst_ref=qr_ref.at[s],
                send_sem=ch_send.at[d], recv_sem=ch_recv.at[s],
                device_id=(s,), device_id_type=pl.DeviceIdType.MESH,
            ).wait_recv()
            out_ref[pl.ds(s * m_per, m_per), :] = (
                qr_ref[s].astype(jnp.float32) * scale
            ).astype(jnp.bfloat16)
        for d in range(1, N_DEV):
            pltpu.make_async_remote_copy(
                src_ref=q_ref.at[0], dst_ref=qr_ref.at[my],
                send_sem=ch_send.at[d], recv_sem=ch_recv.at[my],
                device_id=(my,), device_id_type=pl.DeviceIdType.MESH,
            ).wait_send()

    return pl.pallas_call(
        body,
        out_shape=jax.ShapeDtypeStruct((N_DEV * m_per, n_per), jnp.bfloat16),
        in_specs=[
            pl.BlockSpec(memory_space=pl.ANY),
            pl.BlockSpec(memory_space=pl.ANY),
        ],
        out_specs=pl.BlockSpec(memory_space=pltpu.VMEM),
        scratch_shapes=[
            pltpu.VMEM((m_per, X_CHUNK), jnp.float32),
            pltpu.VMEM((m_per, k), jnp.bfloat16),
            pltpu.VMEM((2, k, W_CHUNK), jnp.float32),
            pltpu.VMEM((N_DEV, m_per, n_per), jnp.float32),
            pltpu.VMEM((N_DEV, m_per, n_per), jnp.int8),
            pltpu.VMEM((N_DEV, m_per, n_per), jnp.int8),
            pltpu.VMEM((N_DEV, 8, 128), jnp.float32),
            pltpu.SemaphoreType.DMA,
            pltpu.SemaphoreType.DMA((2,)),
            pltpu.SemaphoreType.DMA,
            pltpu.SemaphoreType.DMA((N_DEV,)),
            pltpu.SemaphoreType.DMA((N_DEV,)),
            pltpu.SemaphoreType.DMA((N_DEV,)),
            pltpu.SemaphoreType.DMA((N_DEV,)),
        ],
        compiler_params=pltpu.CompilerParams(
            vmem_limit_bytes=58 * 1024 * 1024),
    )(x, w_mat)


# device time: 98666 ns/iter; 1.5057x vs baseline; 1.1603x over previous
import jax
import jax.numpy as jnp
from jax import lax
from jax.experimental import pallas as pl
from jax.experimental.pallas import tpu as pltpu

N_DEV = 8
W_CHUNK = 512
X_CHUNK = 1024


def kernel(x, w_mat):
    m_per, k = x.shape
    _, n = w_mat.shape
    n_per = n // N_DEV
    n_wc = n // W_CHUNK
    n_xc = k // X_CHUNK
    per_dest = n_per // W_CHUNK

    def body(x_hbm, w_hbm, out_ref, xs_ref, xb_ref, wbuf, yb_ref, ybr_ref,
             amax_ref, x_sem, w_sems, own_sem, amax_send, amax_recv,
             ch_send, ch_recv):
        my = lax.axis_index("i")

        def seq_off(i):
            d = 1 + i // per_dest
            t = lax.rem(my + d, N_DEV)
            return t * n_per + (i % per_dest) * W_CHUNK

        def w_copy(i, slot):
            return pltpu.make_async_copy(
                w_hbm.at[:, pl.ds(seq_off(i), W_CHUNK)], wbuf.at[slot],
                w_sems.at[slot])

        w_copy(0, 0).start()

        for c in range(n_xc):
            cp = pltpu.make_async_copy(
                x_hbm.at[:, pl.ds(c * X_CHUNK, X_CHUNK)], xs_ref, x_sem)
            cp.start()
            cp.wait()
            xb_ref[:, pl.ds(c * X_CHUNK, X_CHUNK)] = (
                xs_ref[:, :].astype(jnp.bfloat16))

        local_amax = jnp.float32(0.0)
        for i in range(n_wc):
            slot = i % 2
            if i + 1 < n_wc:
                w_copy(i + 1, (i + 1) % 2).start()
            w_copy(i, slot).wait()
            yj = jnp.dot(xb_ref[:, :], wbuf[slot].astype(jnp.bfloat16),
                         preferred_element_type=jnp.float32)
            d = 1 + i // per_dest
            t = lax.rem(my + d, N_DEV)
            c = i % per_dest
            yb_ref[t, :, pl.ds(c * W_CHUNK, W_CHUNK)] = (
                yj.astype(jnp.bfloat16))
            local_amax = jnp.maximum(local_amax, jnp.max(jnp.abs(yj)))
            if c == per_dest - 1:
                if d < N_DEV:
                    pltpu.make_async_remote_copy(
                        src_ref=yb_ref.at[t], dst_ref=ybr_ref.at[my],
                        send_sem=ch_send.at[d], recv_sem=ch_recv.at[my],
                        device_id=(t,),
                        device_id_type=pl.DeviceIdType.MESH,
                    ).start()
                else:
                    pltpu.make_async_copy(
                        yb_ref.at[my], ybr_ref.at[my], own_sem).start()

        amax_ref[my] = jnp.full((8, 128), local_amax, jnp.float32)
        for d in range(1, N_DEV):
            t = lax.rem(my + d, N_DEV)
            pltpu.make_async_remote_copy(
                src_ref=amax_ref.at[my], dst_ref=amax_ref.at[my],
                send_sem=amax_send.at[d], recv_sem=amax_recv.at[my],
                device_id=(t,), device_id_type=pl.DeviceIdType.MESH,
            ).start()
        for d in range(1, N_DEV):
            s = lax.rem(my + d, N_DEV)
            pltpu.make_async_remote_copy(
                src_ref=amax_ref.at[my], dst_ref=amax_ref.at[s],
                send_sem=amax_send.at[d], recv_sem=amax_recv.at[s],
                device_id=(s,), device_id_type=pl.DeviceIdType.MESH,
            ).wait_recv()
        for d in range(1, N_DEV):
            pltpu.make_async_remote_copy(
                src_ref=amax_ref.at[my], dst_ref=amax_ref.at[my],
                send_sem=amax_send.at[d], recv_sem=amax_recv.at[my],
                device_id=(my,), device_id_type=pl.DeviceIdType.MESH,
            ).wait_send()

        gmax = jnp.max(amax_ref[:, :, :])
        scale = gmax / 127.0
        inv_scale = 127.0 / gmax

        def qdq(s):
            out_ref[pl.ds(s * m_per, m_per), :] = (jnp.clip(
                jnp.round(ybr_ref[s].astype(jnp.float32) * inv_scale),
                -127.0, 127.0) * scale).astype(jnp.bfloat16)

        pltpu.make_async_copy(
            yb_ref.at[my], ybr_ref.at[my], own_sem).wait()
        qdq(my)
        for d in range(1, N_DEV):
            s = lax.rem(my + d, N_DEV)
            pltpu.make_async_remote_copy(
                src_ref=yb_ref.at[0], dst_ref=ybr_ref.at[s],
                send_sem=ch_send.at[d], recv_sem=ch_recv.at[s],
                device_id=(s,), device_id_type=pl.DeviceIdType.MESH,
            ).wait_recv()
            qdq(s)
        for d in range(1, N_DEV):
            pltpu.make_async_remote_copy(
                src_ref=yb_ref.at[0], dst_ref=ybr_ref.at[my],
                send_sem=ch_send.at[d], recv_sem=ch_recv.at[my],
                device_id=(my,), device_id_type=pl.DeviceIdType.MESH,
            ).wait_send()

    return pl.pallas_call(
        body,
        out_shape=jax.ShapeDtypeStruct((N_DEV * m_per, n_per), jnp.bfloat16),
        in_specs=[
            pl.BlockSpec(memory_space=pl.ANY),
            pl.BlockSpec(memory_space=pl.ANY),
        ],
        out_specs=pl.BlockSpec(memory_space=pltpu.VMEM),
        scratch_shapes=[
            pltpu.VMEM((m_per, X_CHUNK), jnp.float32),
            pltpu.VMEM((m_per, k), jnp.bfloat16),
            pltpu.VMEM((2, k, W_CHUNK), jnp.float32),
            pltpu.VMEM((N_DEV, m_per, n_per), jnp.bfloat16),
            pltpu.VMEM((N_DEV, m_per, n_per), jnp.bfloat16),
            pltpu.VMEM((N_DEV, 8, 128), jnp.float32),
            pltpu.SemaphoreType.DMA,
            pltpu.SemaphoreType.DMA((2,)),
            pltpu.SemaphoreType.DMA,
            pltpu.SemaphoreType.DMA((N_DEV,)),
            pltpu.SemaphoreType.DMA((N_DEV,)),
            pltpu.SemaphoreType.DMA((N_DEV,)),
            pltpu.SemaphoreType.DMA((N_DEV,)),
        ],
        compiler_params=pltpu.CompilerParams(
            vmem_limit_bytes=58 * 1024 * 1024),
    )(x, w_mat)
